# Initial kernel scaffold; baseline (speedup 1.0000x reference)
import jax
import jax.numpy as jnp
from jax.experimental import pallas as pl


def kernel(x, W, b):
    raise NotImplementedError("write your pallas kernel here")


# speedup vs baseline: 2.3059x; 2.3059x over previous
import functools

import jax
import jax.numpy as jnp
import numpy as np
from jax.experimental import pallas as pl

BS = 128
N = 2048
C = 32
Q = 64
K = 5
F = N // 2 + 1
FP = 1032
BIG = 1e9


def _dft_tables():
    t = np.arange(N)[None, :]
    f = np.arange(F)[:, None]
    ang = 2.0 * np.pi * f * t / N
    cs = np.zeros((2 * FP, N), dtype=np.float32)
    cs[:F, :] = np.cos(ang)
    cs[FP:FP + F, :] = -np.sin(ang)
    return jnp.asarray(cs)


def _router_kernel(cs_ref, xt_ref, wt_ref, b_ref, g_ref, out_ref):
    c = pl.program_id(0)
    nsteps = pl.num_programs(0)

    xspec = jax.lax.dot(
        cs_ref[...], xt_ref[...],
        precision=jax.lax.Precision.HIGHEST,
        preferred_element_type=jnp.float32)
    wspec = jax.lax.dot(
        cs_ref[...], wt_ref[0],
        precision=jax.lax.Precision.HIGHEST,
        preferred_element_type=jnp.float32)

    xr = xspec[:FP, :]
    xi = xspec[FP:, :]
    mag2 = xr * xr + xi * xi

    iota_f = jax.lax.broadcasted_iota(jnp.int32, (FP, BS), 0)
    sel = jnp.zeros((FP, BS), dtype=jnp.bool_)
    for _ in range(K):
        m = jnp.max(mag2, axis=0, keepdims=True)
        amax = jnp.min(jnp.where(mag2 == m, iota_f, jnp.int32(2**30)),
                       axis=0, keepdims=True)
        pick = iota_f == amax
        sel = jnp.logical_or(sel, pick)
        mag2 = jnp.where(pick, -1.0, mag2)

    alpha = jnp.where(
        jnp.logical_or(iota_f == 0, iota_f == N // 2),
        jnp.float32(1.0 / N), jnp.float32(2.0 / N))
    ar = jnp.where(sel, xr * alpha, 0.0)
    ai = jnp.where(sel, xi * alpha, 0.0)

    dn = (((0,), (0,)), ((), ()))
    acc = jax.lax.dot_general(
        ar, wspec[:FP, :], dn,
        precision=jax.lax.Precision.HIGHEST,
        preferred_element_type=jnp.float32)
    acc = acc + jax.lax.dot_general(
        ai, wspec[FP:, :], dn,
        precision=jax.lax.Precision.HIGHEST,
        preferred_element_type=jnp.float32)

    @pl.when(c == 0)
    def _init():
        out_ref[...] = acc

    @pl.when(c > 0)
    def _accum():
        out_ref[...] += acc

    @pl.when(c == nsteps - 1)
    def _epilogue():
        z = out_ref[...] + b_ref[...] + g_ref[...]
        m = jnp.max(z, axis=1, keepdims=True)
        e = jnp.exp(z - m)
        y = e / jnp.sum(e, axis=1, keepdims=True)
        iota_q = jax.lax.broadcasted_iota(jnp.int32, (BS, Q), 1)
        za = jnp.max(z, axis=1, keepdims=True)
        first = jnp.min(jnp.where(z == za, iota_q, jnp.int32(2**30)),
                        axis=1, keepdims=True)
        hard = jnp.where(iota_q == first, jnp.float32(1.0), jnp.float32(0.0))
        out_ref[...] = (hard - y) + y


@functools.partial(jax.jit, static_argnames=())
def _run(x, W, b, cs, g):
    xt = jnp.transpose(x, (1, 2, 0)).reshape(N, C * BS)
    wt = jnp.transpose(W.reshape(Q, N, C), (2, 1, 0))
    bb = b.reshape(1, Q)

    grid = (C,)
    return pl.pallas_call(
        _router_kernel,
        grid=grid,
        in_specs=[
            pl.BlockSpec((2 * FP, N), lambda i: (0, 0)),
            pl.BlockSpec((N, BS), lambda i: (0, i)),
            pl.BlockSpec((1, N, Q), lambda i: (i, 0, 0)),
            pl.BlockSpec((1, Q), lambda i: (0, 0)),
            pl.BlockSpec((BS, Q), lambda i: (0, 0)),
        ],
        out_specs=pl.BlockSpec((BS, Q), lambda i: (0, 0)),
        out_shape=jax.ShapeDtypeStruct((BS, Q), jnp.float32),
    )(cs, xt, wt, bb, g)


def kernel(x, W, b):
    cs = _dft_tables()
    g = jax.random.gumbel(jax.random.key(42), (BS, Q), dtype=jnp.float32)
    return _run(x, W, b, cs, g)


# speedup vs baseline: 6.3613x; 2.7587x over previous
import functools

import jax
import jax.numpy as jnp
import numpy as np
from jax.experimental import pallas as pl

BS = 128
N = 2048
C = 32
Q = 64
K = 5
F = N // 2 + 1
FP = 1032
CPS = 2


def _dft_tables():
    t = np.arange(N)[None, :]
    f = np.arange(F)[:, None]
    ang = 2.0 * np.pi * f * t / N
    cs = np.zeros((2 * FP, N), dtype=np.float32)
    cs[:F, :] = np.cos(ang)
    cs[FP:FP + F, :] = -np.sin(ang)
    return jnp.asarray(cs)


def _split(v):
    hi = v.astype(jnp.bfloat16)
    lo = (v - hi.astype(jnp.float32)).astype(jnp.bfloat16)
    return hi, lo


def _dot3(ah, al, bh, bl, dn):
    d = functools.partial(
        jax.lax.dot_general, dimension_numbers=dn,
        preferred_element_type=jnp.float32)
    return d(ah, bh) + (d(ah, bl) + d(al, bh))


_MN = (((1,), (0,)), ((), ()))
_TN = (((0,), (0,)), ((), ()))


def _wspec_kernel(csh_ref, csl_ref, wt_ref, wr_ref):
    wh, wl = _split(wt_ref[...])
    wr_ref[...] = _dot3(csh_ref[...], csl_ref[...], wh, wl, _MN)


def _router_kernel(csh_ref, csl_ref, xt_ref, wr_ref, b_ref, g_ref, out_ref):
    i = pl.program_id(0)
    nsteps = pl.num_programs(0)

    xh, xl = _split(xt_ref[...])
    xspec = _dot3(csh_ref[...], csl_ref[...], xh, xl, _MN)

    xr = xspec[:FP, :]
    xi = xspec[FP:, :]
    mag2 = xr * xr + xi * xi

    iota_f = jax.lax.broadcasted_iota(jnp.int32, (FP, CPS * BS), 0)
    sel = jnp.zeros((FP, CPS * BS), dtype=jnp.bool_)
    for _ in range(K):
        m = jnp.max(mag2, axis=0, keepdims=True)
        amax = jnp.min(jnp.where(mag2 == m, iota_f, jnp.int32(2**30)),
                       axis=0, keepdims=True)
        pick = iota_f == amax
        sel = jnp.logical_or(sel, pick)
        mag2 = jnp.where(pick, -1.0, mag2)

    alpha = jnp.where(
        jnp.logical_or(iota_f == 0, iota_f == N // 2),
        jnp.float32(1.0 / N), jnp.float32(2.0 / N))
    ar = jnp.where(sel, xr * alpha, 0.0)
    ai = jnp.where(sel, xi * alpha, 0.0)

    wr = wr_ref[...]
    acc = jnp.zeros((BS, Q), dtype=jnp.float32)
    for j in range(CPS):
        bcols = slice(j * BS, (j + 1) * BS)
        qcols = slice(j * Q, (j + 1) * Q)
        arh, arl = _split(ar[:, bcols])
        aih, ail = _split(ai[:, bcols])
        wrh, wrl = _split(wr[:FP, qcols])
        wih, wil = _split(wr[FP:, qcols])
        acc = acc + _dot3(arh, arl, wrh, wrl, _TN)
        acc = acc + _dot3(aih, ail, wih, wil, _TN)

    @pl.when(i == 0)
    def _init():
        out_ref[...] = acc

    @pl.when(i > 0)
    def _accum():
        out_ref[...] += acc

    @pl.when(i == nsteps - 1)
    def _epilogue():
        z = out_ref[...] + b_ref[...] + g_ref[...]
        m = jnp.max(z, axis=1, keepdims=True)
        e = jnp.exp(z - m)
        y = e / jnp.sum(e, axis=1, keepdims=True)
        iota_q = jax.lax.broadcasted_iota(jnp.int32, (BS, Q), 1)
        first = jnp.min(jnp.where(z == m, iota_q, jnp.int32(2**30)),
                        axis=1, keepdims=True)
        hard = jnp.where(iota_q == first, jnp.float32(1.0), jnp.float32(0.0))
        out_ref[...] = (hard - y) + y


@jax.jit
def _run(x, W, b, csh, csl, g):
    xt = jnp.transpose(x, (1, 2, 0)).reshape(N, C * BS)
    wt = jnp.transpose(W.reshape(Q, N, C), (1, 2, 0)).reshape(N, C * Q)
    bb = b.reshape(1, Q)

    wr = pl.pallas_call(
        _wspec_kernel,
        grid=(4,),
        in_specs=[
            pl.BlockSpec((2 * FP, N), lambda i: (0, 0)),
            pl.BlockSpec((2 * FP, N), lambda i: (0, 0)),
            pl.BlockSpec((N, C * Q // 4), lambda i: (0, i)),
        ],
        out_specs=pl.BlockSpec((2 * FP, C * Q // 4), lambda i: (0, i)),
        out_shape=jax.ShapeDtypeStruct((2 * FP, C * Q), jnp.float32),
    )(csh, csl, wt)

    return pl.pallas_call(
        _router_kernel,
        grid=(C // CPS,),
        in_specs=[
            pl.BlockSpec((2 * FP, N), lambda i: (0, 0)),
            pl.BlockSpec((2 * FP, N), lambda i: (0, 0)),
            pl.BlockSpec((N, CPS * BS), lambda i: (0, i)),
            pl.BlockSpec((2 * FP, CPS * Q), lambda i: (0, i)),
            pl.BlockSpec((1, Q), lambda i: (0, 0)),
            pl.BlockSpec((BS, Q), lambda i: (0, 0)),
        ],
        out_specs=pl.BlockSpec((BS, Q), lambda i: (0, 0)),
        out_shape=jax.ShapeDtypeStruct((BS, Q), jnp.float32),
    )(csh, csl, xt, wr, bb, g)


def kernel(x, W, b):
    csh, csl = _split(_dft_tables())
    g = jax.random.gumbel(jax.random.key(42), (BS, Q), dtype=jnp.float32)
    return _run(x, W, b, csh, csl, g)
